# Initial kernel scaffold; baseline (speedup 1.0000x reference)
import jax
import jax.numpy as jnp
from jax.experimental import pallas as pl


def kernel(x, edge_index, W, b):
    raise NotImplementedError("write your pallas kernel here")


# speedup vs baseline: 5.6700x; 5.6700x over previous
import functools

import jax
import jax.numpy as jnp
from jax import lax
from jax.experimental import pallas as pl
from jax.experimental.pallas import tpu as pltpu
from jax.experimental.pallas import tpu_sc as plsc

_NC, _NS, _L = 2, 16, 16
_NW = _NC * _NS


def _sc_maxrel(xT_pad, eidx_pad, npad, c):
    chunk = npad // _NW
    k2 = eidx_pad.shape[1]
    kk = k2 // 2
    grp = c // _L
    mesh = plsc.VectorSubcoreMesh(core_axis_name="c", subcore_axis_name="s")

    @functools.partial(
        pl.kernel,
        out_type=jax.ShapeDtypeStruct((npad, c), jnp.float32),
        mesh=mesh,
        scratch_types=[
            pltpu.VMEM((chunk, k2), jnp.int32),
            pltpu.VMEM((k2, c), jnp.float32),
            pltpu.VMEM((chunk, c), jnp.float32),
            pltpu.SemaphoreType.DMA,
        ],
    )
    def body(xT_hbm, eidx_hbm, out_hbm, idx_v, rows_v, out_v, sem):
        wid = lax.axis_index("s") * _NC + lax.axis_index("c")
        base = wid * chunk
        pltpu.sync_copy(eidx_hbm.at[pl.ds(base, chunk)], idx_v)

        def node_body(t, carry):
            pltpu.async_copy(xT_hbm.at[idx_v.at[t]], rows_v, sem).wait()

            def k_body(k, acc):
                return tuple(
                    jnp.maximum(
                        acc[g],
                        rows_v[k, pl.ds(_L * g, _L)]
                        - rows_v[kk + k, pl.ds(_L * g, _L)],
                    )
                    for g in range(grp)
                )

            acc0 = tuple(
                rows_v[0, pl.ds(_L * g, _L)] - rows_v[kk, pl.ds(_L * g, _L)]
                for g in range(grp)
            )
            acc = lax.fori_loop(1, kk, k_body, acc0)
            for g in range(grp):
                out_v[t, pl.ds(_L * g, _L)] = acc[g]
            return carry

        lax.fori_loop(0, chunk, node_body, 0)
        pltpu.sync_copy(out_v, out_hbm.at[pl.ds(base, chunk)])

    return body(xT_pad, eidx_pad)


def _tc_conv(xT_pad, y_pad, waT, wbT, brow, n):
    c = xT_pad.shape[1]
    out_c = waT.shape[1]
    bn = 1024

    def body(xT_ref, y_ref, waT_ref, wbT_ref, b_ref, o_ref):
        acc = jnp.dot(xT_ref[...], waT_ref[...], preferred_element_type=jnp.float32)
        acc = acc + jnp.dot(y_ref[...], wbT_ref[...], preferred_element_type=jnp.float32)
        o_ref[...] = jnp.maximum(acc + b_ref[...], 0.0)

    return pl.pallas_call(
        body,
        grid=(pl.cdiv(n, bn),),
        in_specs=[
            pl.BlockSpec((bn, c), lambda i: (i, 0)),
            pl.BlockSpec((bn, c), lambda i: (i, 0)),
            pl.BlockSpec((c, out_c), lambda i: (0, 0)),
            pl.BlockSpec((c, out_c), lambda i: (0, 0)),
            pl.BlockSpec((1, out_c), lambda i: (0, 0)),
        ],
        out_specs=pl.BlockSpec((bn, out_c), lambda i: (i, 0)),
        out_shape=jax.ShapeDtypeStruct((n, out_c), jnp.float32),
    )(xT_pad, y_pad, waT, wbT, brow)


def kernel(x, edge_index, W, b):
    _, c, n = x.shape
    npad = -(-n // (8 * _NW)) * (8 * _NW)
    xT = jnp.pad(x[0].T, ((0, npad - n), (0, 0)))
    e0 = edge_index[0, 0]
    e1 = edge_index[1, 0]
    eidx = jnp.pad(
        jnp.concatenate([e0, e1], axis=1), ((0, npad - n), (0, 0))
    )
    y_pad = _sc_maxrel(xT, eidx, npad, c)

    w2 = W[:, :, 0]
    waT = w2[:, :c].T
    wbT = w2[:, c:].T
    outT = _tc_conv(xT, y_pad, waT, wbT, b[None, :], n)
    return jnp.transpose(outT)[None]


# speedup vs baseline: 7.3581x; 1.2977x over previous
import functools

import jax
import jax.numpy as jnp
from jax import lax
from jax.experimental import pallas as pl
from jax.experimental.pallas import tpu as pltpu
from jax.experimental.pallas import tpu_sc as plsc

_NC, _NS, _L = 2, 16, 16
_NW = _NC * _NS


def _sc_maxrel(xT_pad, eidx_pad, npad, c):
    chunk = npad // _NW
    k2 = eidx_pad.shape[1]
    kk = k2 // 4
    grp = c // _L
    nb = chunk // 2
    mesh = plsc.VectorSubcoreMesh(core_axis_name="c", subcore_axis_name="s")

    @functools.partial(
        pl.kernel,
        out_type=jax.ShapeDtypeStruct((npad, c), jnp.float32),
        mesh=mesh,
        scratch_types=[
            pltpu.VMEM((nb, k2), jnp.int32),
            pltpu.VMEM((2, k2, c), jnp.float32),
            pltpu.VMEM((chunk, c), jnp.float32),
            pltpu.SemaphoreType.DMA,
            pltpu.SemaphoreType.DMA,
        ],
    )
    def body(xT_hbm, eidx_hbm, out_hbm, idx_v, rows_v, out_v, sem0, sem1):
        wid = lax.axis_index("s") * _NC + lax.axis_index("c")
        base = wid * chunk
        pltpu.sync_copy(eidx_hbm.at[pl.ds(wid * nb, nb)], idx_v)
        sems = (sem0, sem1)

        def start(b, slot):
            pltpu.async_copy(xT_hbm.at[idx_v.at[b]], rows_v.at[slot], sems[slot])

        def wait(b, slot):
            pltpu.make_async_copy(
                xT_hbm.at[idx_v.at[b]], rows_v.at[slot], sems[slot]
            ).wait()

        def compute(b, slot):
            for j in range(2):
                off = 2 * kk * j

                def k_body(k, acc):
                    return tuple(
                        jnp.maximum(
                            acc[g],
                            rows_v[slot, off + k, pl.ds(_L * g, _L)]
                            - rows_v[slot, off + kk + k, pl.ds(_L * g, _L)],
                        )
                        for g in range(grp)
                    )

                acc = lax.fori_loop(
                    1,
                    kk,
                    k_body,
                    tuple(
                        rows_v[slot, off, pl.ds(_L * g, _L)]
                        - rows_v[slot, off + kk, pl.ds(_L * g, _L)]
                        for g in range(grp)
                    ),
                )
                t = 2 * b + j
                for g in range(grp):
                    out_v[t, pl.ds(_L * g, _L)] = acc[g]

        start(0, 0)

        def pair_body(i, carry):
            b0 = 2 * i
            start(b0 + 1, 1)
            wait(b0, 0)
            compute(b0, 0)

            @pl.when(b0 + 2 < nb)
            def _():
                start(b0 + 2, 0)

            wait(b0 + 1, 1)
            compute(b0 + 1, 1)
            return carry

        lax.fori_loop(0, nb // 2, pair_body, 0)
        pltpu.sync_copy(out_v, out_hbm.at[pl.ds(base, chunk)])

    return body(xT_pad, eidx_pad)


def _tc_conv(xT_pad, y_pad, waT, wbT, brow, n):
    c = xT_pad.shape[1]
    out_c = waT.shape[1]
    bn = 1024

    def body(xT_ref, y_ref, waT_ref, wbT_ref, b_ref, o_ref):
        acc = jnp.dot(xT_ref[...], waT_ref[...], preferred_element_type=jnp.float32)
        acc = acc + jnp.dot(y_ref[...], wbT_ref[...], preferred_element_type=jnp.float32)
        o_ref[...] = jnp.maximum(acc + b_ref[...], 0.0)

    return pl.pallas_call(
        body,
        grid=(pl.cdiv(n, bn),),
        in_specs=[
            pl.BlockSpec((bn, c), lambda i: (i, 0)),
            pl.BlockSpec((bn, c), lambda i: (i, 0)),
            pl.BlockSpec((c, out_c), lambda i: (0, 0)),
            pl.BlockSpec((c, out_c), lambda i: (0, 0)),
            pl.BlockSpec((1, out_c), lambda i: (0, 0)),
        ],
        out_specs=pl.BlockSpec((bn, out_c), lambda i: (i, 0)),
        out_shape=jax.ShapeDtypeStruct((n, out_c), jnp.float32),
    )(xT_pad, y_pad, waT, wbT, brow)


def kernel(x, edge_index, W, b):
    _, c, n = x.shape
    npad = -(-n // (8 * _NW)) * (8 * _NW)
    xT = jnp.pad(x[0].T, ((0, npad - n), (0, 0)))
    e0 = edge_index[0, 0]
    e1 = edge_index[1, 0]
    eidx = jnp.pad(
        jnp.concatenate([e0, e1], axis=1), ((0, npad - n), (0, 0))
    ).reshape(npad // 2, -1)
    y_pad = _sc_maxrel(xT, eidx, npad, c)

    w2 = W[:, :, 0]
    waT = w2[:, :c].T
    wbT = w2[:, c:].T
    outT = _tc_conv(xT, y_pad, waT, wbT, b[None, :], n)
    return jnp.transpose(outT)[None]


# speedup vs baseline: 31.4016x; 4.2676x over previous
import functools

import jax
import jax.numpy as jnp
from jax import lax
from jax.experimental import pallas as pl
from jax.experimental.pallas import tpu as pltpu
from jax.experimental.pallas import tpu_sc as plsc

_NC, _NS, _L = 2, 16, 16
_NW = _NC * _NS


def _sc_maxrel(xT_pad, eidx_pad, npad, c):
    chunk = npad // _NW
    k2 = eidx_pad.shape[1]
    kk = k2 // 2
    grp = c // _L
    outb = 16
    nblk = chunk // outb
    mesh = plsc.VectorSubcoreMesh(core_axis_name="c", subcore_axis_name="s")

    @functools.partial(
        pl.kernel,
        out_type=jax.ShapeDtypeStruct((npad, c), jnp.float32),
        mesh=mesh,
        scratch_types=[
            pltpu.VMEM((2, outb, k2), jnp.int32),
            pltpu.VMEM((2, k2, c), jnp.float32),
            pltpu.VMEM((2, outb, c), jnp.float32),
            pltpu.VMEM_SHARED((npad, c), jnp.float32),
            pltpu.SemaphoreType.DMA,
            pltpu.SemaphoreType.DMA,
            pltpu.SemaphoreType.DMA,
            pltpu.SemaphoreType.DMA,
            pltpu.SemaphoreType.DMA,
            pltpu.SemaphoreType.DMA,
        ],
    )
    def body(xT_hbm, eidx_hbm, out_hbm, idx_v, rows_v, out_v, tbl_s,
             gsem0, gsem1, osem0, osem1, isem0, isem1):
        sid = lax.axis_index("s")
        wid = sid * _NC + lax.axis_index("c")
        base = wid * chunk
        stg = npad // _NS
        pltpu.sync_copy(
            xT_hbm.at[pl.ds(sid * stg, stg)], tbl_s.at[pl.ds(sid * stg, stg)]
        )
        plsc.subcore_barrier()
        gsems = (gsem0, gsem1)
        osems = (osem0, osem1)
        isems = (isem0, isem1)

        def istart(bk, islot):
            pltpu.async_copy(
                eidx_hbm.at[pl.ds(base + bk * outb, outb)],
                idx_v.at[islot],
                isems[islot],
            )

        def iwait(islot):
            pltpu.make_async_copy(
                eidx_hbm.at[pl.ds(base, outb)], idx_v.at[islot], isems[islot]
            ).wait()

        def start(loc, gslot, islot):
            pltpu.async_copy(
                tbl_s.at[idx_v.at[islot, loc]], rows_v.at[gslot], gsems[gslot]
            )

        def wait(gslot):
            pltpu.make_async_copy(
                tbl_s.at[idx_v.at[0, 0]], rows_v.at[gslot], gsems[gslot]
            ).wait()

        def compute(gslot, oslot, loc):
            def k_body(k, acc):
                return tuple(
                    jnp.maximum(
                        acc[g],
                        rows_v[gslot, k, pl.ds(_L * g, _L)]
                        - rows_v[gslot, kk + k, pl.ds(_L * g, _L)],
                    )
                    for g in range(grp)
                )

            acc = lax.fori_loop(
                1,
                kk,
                k_body,
                tuple(
                    rows_v[gslot, 0, pl.ds(_L * g, _L)]
                    - rows_v[gslot, kk, pl.ds(_L * g, _L)]
                    for g in range(grp)
                ),
            )
            for g in range(grp):
                out_v[oslot, loc, pl.ds(_L * g, _L)] = acc[g]

        def oscatter(bk, oslot):
            pltpu.async_copy(
                out_v.at[oslot],
                out_hbm.at[pl.ds(base + bk * outb, outb)],
                osems[oslot],
            )

        def owait(oslot):
            pltpu.make_async_copy(
                out_v.at[oslot], out_hbm.at[pl.ds(base, outb)], osems[oslot]
            ).wait()

        istart(0, 0)

        def block(bk, oslot):
            @pl.when(bk + 1 < nblk)
            def _():
                istart(bk + 1, 1 - oslot)

            iwait(oslot)

            @pl.when(bk >= 2)
            def _():
                owait(oslot)

            start(0, 0, oslot)

            def pair_body(i, carry):
                l0 = 2 * i
                start(l0 + 1, 1, oslot)
                wait(0)
                compute(0, oslot, l0)

                @pl.when(l0 + 2 < outb)
                def _():
                    start(l0 + 2, 0, oslot)

                wait(1)
                compute(1, oslot, l0 + 1)
                return carry

            lax.fori_loop(0, outb // 2, pair_body, 0)
            oscatter(bk, oslot)

        def outer(m, carry):
            block(2 * m, 0)
            block(2 * m + 1, 1)
            return carry

        lax.fori_loop(0, nblk // 2, outer, 0)
        owait(0)
        owait(1)

    return body(xT_pad, eidx_pad)


def _tc_conv(xT_pad, y_pad, waT, wbT, brow, n):
    c = xT_pad.shape[1]
    out_c = waT.shape[1]
    bn = 1024

    def body(xT_ref, y_ref, waT_ref, wbT_ref, b_ref, o_ref):
        acc = jnp.dot(xT_ref[...], waT_ref[...], preferred_element_type=jnp.float32)
        acc = acc + jnp.dot(y_ref[...], wbT_ref[...], preferred_element_type=jnp.float32)
        o_ref[...] = jnp.maximum(acc + b_ref[...], 0.0)

    return pl.pallas_call(
        body,
        grid=(pl.cdiv(n, bn),),
        in_specs=[
            pl.BlockSpec((bn, c), lambda i: (i, 0)),
            pl.BlockSpec((bn, c), lambda i: (i, 0)),
            pl.BlockSpec((c, out_c), lambda i: (0, 0)),
            pl.BlockSpec((c, out_c), lambda i: (0, 0)),
            pl.BlockSpec((1, out_c), lambda i: (0, 0)),
        ],
        out_specs=pl.BlockSpec((bn, out_c), lambda i: (i, 0)),
        out_shape=jax.ShapeDtypeStruct((n, out_c), jnp.float32),
    )(xT_pad, y_pad, waT, wbT, brow)


def kernel(x, edge_index, W, b):
    _, c, n = x.shape
    npad = -(-n // (8 * _NW)) * (8 * _NW)
    xT = jnp.pad(x[0].T, ((0, npad - n), (0, 0)))
    e0 = edge_index[0, 0]
    e1 = edge_index[1, 0]
    eidx = jnp.pad(
        jnp.concatenate([e0, e1], axis=1), ((0, npad - n), (0, 0))
    )
    y_pad = _sc_maxrel(xT, eidx, npad, c)

    w2 = W[:, :, 0]
    waT = w2[:, :c].T
    wbT = w2[:, c:].T
    outT = _tc_conv(xT, y_pad, waT, wbT, b[None, :], n)
    return jnp.transpose(outT)[None]
